# baseline (speedup 1.0000x reference)
import functools

import jax
import jax.numpy as jnp
from jax import lax
from jax.experimental import pallas as pl
from jax.experimental.pallas import tpu as pltpu
from jax.experimental.pallas import tpu_sc as plsc

_N, _E, _D, _H, _B, _C = 10000, 320000, 128, 128, 8, 2
_NC, _NS = 2, 16
_NW = _NC * _NS
_CH = 128
_EPT = _E // _NW
_NCHUNK = 79
_EPAD = _NCHUNK * _CH
_NACC = _N + 8
_RPT = 624
_ZR = _CH
_RTAIL = _N - _NS * _RPT

_PPT = 312
_PCH = 104
_PTAIL = _N - _NW * _PPT

_sc_mesh = plsc.VectorSubcoreMesh(core_axis_name="c", subcore_axis_name="s")


def _edge_aggr_body(h_hbm, idx_hbm, zeros_hbm, out_hbm,
                    acc_sh, idx_v, rows, gsems, ssems):
    c = lax.axis_index("c")
    s = lax.axis_index("s")
    wid = c * _NS + s

    pltpu.sync_copy(zeros_hbm, rows[0])
    for k in range(4):
        pltpu.sync_copy(rows[0], acc_sh.at[pl.ds(s * _RPT + k * _CH, _CH)])
    pltpu.sync_copy(rows[0].at[pl.ds(0, 112)],
                    acc_sh.at[pl.ds(s * _RPT + 4 * _CH, 112)])

    @pl.when(s == 0)
    def _zero_tail():
        pltpu.sync_copy(rows[0].at[pl.ds(0, _RTAIL)],
                        acc_sh.at[pl.ds(_NS * _RPT, _RTAIL)])

    plsc.subcore_barrier()

    def gather(j, b):
        pltpu.async_copy(h_hbm.at[idx_v.at[2 * j]], rows[b], gsems[b])

    def gather_wait(j, b):
        pltpu.make_async_copy(h_hbm.at[idx_v.at[2 * j]], rows[b],
                              gsems[b]).wait()

    def scatter(j, b):
        pltpu.async_copy(rows[b], acc_sh.at[idx_v.at[2 * j + 1]], ssems[b],
                         add=True)

    def scatter_wait(j, b):
        pltpu.make_async_copy(rows[b], acc_sh.at[idx_v.at[2 * j + 1]],
                              ssems[b]).wait()

    def run_phase(nchunks):
        npairs = (nchunks - 2) // 2

        def body(k, carry):
            a = 2 * k
            gather_wait(a, 0)
            scatter(a, 0)
            gather_wait(a + 1, 1)
            scatter(a + 1, 1)
            scatter_wait(a, 0)
            gather(a + 2, 0)
            scatter_wait(a + 1, 1)
            gather(a + 3, 1)
            return carry

        gather(0, 0)
        gather(1, 1)
        lax.fori_loop(0, npairs, body, 0)
        a = 2 * npairs
        gather_wait(a, 0)
        scatter(a, 0)
        gather_wait(a + 1, 1)
        scatter(a + 1, 1)
        scatter_wait(a, 0)
        if nchunks % 2:
            gather(a + 2, 0)
            gather_wait(a + 2, 0)
            scatter(a + 2, 0)
            scatter_wait(a + 2, 0)
        scatter_wait(a + 1, 1)

    pltpu.sync_copy(idx_hbm.at[wid, pl.ds(0, 80)], idx_v)
    run_phase(40)
    pltpu.sync_copy(idx_hbm.at[wid, pl.ds(80, 78)], idx_v.at[pl.ds(0, 78)])
    run_phase(39)

    plsc.subcore_barrier()

    for k in range(4):
        r0 = s * _RPT + k * _CH
        pltpu.sync_copy(acc_sh.at[pl.ds(r0, _CH)], rows[0])
        pltpu.sync_copy(rows[0], out_hbm.at[c, pl.ds(r0, _CH)])
    r1 = s * _RPT + 4 * _CH
    pltpu.sync_copy(acc_sh.at[pl.ds(r1, 112)], rows[0].at[pl.ds(0, 112)])
    pltpu.sync_copy(rows[0].at[pl.ds(0, 112)], out_hbm.at[c, pl.ds(r1, 112)])

    @pl.when(s == 0)
    def _out_tail():
        r0 = _NS * _RPT
        pltpu.sync_copy(acc_sh.at[pl.ds(r0, _RTAIL)],
                        rows[0].at[pl.ds(0, _RTAIL)])
        pltpu.sync_copy(rows[0].at[pl.ds(0, _RTAIL)],
                        out_hbm.at[c, pl.ds(r0, _RTAIL)])


@functools.partial(
    pl.kernel,
    out_type=jax.ShapeDtypeStruct((_NC, _N, _H), jnp.float32),
    mesh=_sc_mesh,
    scratch_types=[
        pltpu.VMEM_SHARED((_NACC, _H), jnp.float32),
        pltpu.VMEM((80, _CH), jnp.int32),
        pltpu.VMEM((_CH, _H), jnp.float32),
        pltpu.VMEM((_CH, _H), jnp.float32),
        pltpu.SemaphoreType.DMA,
        pltpu.SemaphoreType.DMA,
        pltpu.SemaphoreType.DMA,
        pltpu.SemaphoreType.DMA,
    ],
)
def _edge_aggr(h_hbm, idx_hbm, zeros_hbm, out_hbm,
               acc_sh, idx_v, rows0, rows1, g0, g1, s0, s1):
    _edge_aggr_body(h_hbm, idx_hbm, zeros_hbm, out_hbm,
                    acc_sh, idx_v, (rows0, rows1), (g0, g1), (s0, s1))


def _pool_body(h_hbm, a0_hbm, a1_hbm, batch_hbm, zeros_hbm, out_hbm,
               g_sh, a0_v, a1_v, idxb_v, rows_v, zg_v,
               a0t_v, a1t_v, idxbt_v, rowst_v, sem, sem2):
    c = lax.axis_index("c")
    s = lax.axis_index("s")
    wid = c * _NS + s

    @pl.when(s == 0)
    def _init():
        pltpu.sync_copy(zeros_hbm.at[pl.ds(0, _B)], zg_v)
        pltpu.sync_copy(zg_v, g_sh)

    plsc.subcore_barrier()

    base = wid * _PPT
    for j in range(_PPT // _PCH):
        off = base + j * _PCH
        pltpu.sync_copy(a0_hbm.at[pl.ds(off, _PCH)], a0_v)
        pltpu.sync_copy(a1_hbm.at[pl.ds(off, _PCH)], a1_v)
        cp_rows = pltpu.async_copy(h_hbm.at[a1_v], rows_v, sem)
        cp_idx = pltpu.async_copy(batch_hbm.at[a0_v], idxb_v, sem2)
        cp_rows.wait()
        cp_idx.wait()
        pltpu.sync_copy(rows_v, g_sh.at[idxb_v], add=True)

    @pl.when(wid == 0)
    def _tail():
        off = _NW * _PPT
        pltpu.sync_copy(a0_hbm.at[pl.ds(off, _PTAIL)], a0t_v)
        pltpu.sync_copy(a1_hbm.at[pl.ds(off, _PTAIL)], a1t_v)
        cp_rows = pltpu.async_copy(h_hbm.at[a1t_v], rowst_v, sem)
        cp_idx = pltpu.async_copy(batch_hbm.at[a0t_v], idxbt_v, sem2)
        cp_rows.wait()
        cp_idx.wait()
        pltpu.sync_copy(rowst_v, g_sh.at[idxbt_v], add=True)

    plsc.subcore_barrier()

    @pl.when(s == 0)
    def _out():
        pltpu.sync_copy(g_sh, zg_v)
        pltpu.sync_copy(zg_v, out_hbm.at[c])


@functools.partial(
    pl.kernel,
    out_type=jax.ShapeDtypeStruct((_NC, _B, _H), jnp.float32),
    mesh=_sc_mesh,
    scratch_types=[
        pltpu.VMEM_SHARED((_B, _H), jnp.float32),
        pltpu.VMEM((_PCH,), jnp.int32),
        pltpu.VMEM((_PCH,), jnp.int32),
        pltpu.VMEM((_PCH,), jnp.int32),
        pltpu.VMEM((_PCH, _H), jnp.float32),
        pltpu.VMEM((_B, _H), jnp.float32),
        pltpu.VMEM((_PTAIL,), jnp.int32),
        pltpu.VMEM((_PTAIL,), jnp.int32),
        pltpu.VMEM((_PTAIL,), jnp.int32),
        pltpu.VMEM((_PTAIL, _H), jnp.float32),
        pltpu.SemaphoreType.DMA,
        pltpu.SemaphoreType.DMA,
    ],
)
def _pool(h_hbm, a0_hbm, a1_hbm, batch_hbm, zeros_hbm, out_hbm,
          g_sh, a0_v, a1_v, idxb_v, rows_v, zg_v,
          a0t_v, a1t_v, idxbt_v, rowst_v, sem, sem2):
    _pool_body(h_hbm, a0_hbm, a1_hbm, batch_hbm, zeros_hbm, out_hbm,
               g_sh, a0_v, a1_v, idxb_v, rows_v, zg_v,
               a0t_v, a1t_v, idxbt_v, rowst_v, sem, sem2)


_ROWS = 1000


def _enc_block(x_ref, w_ref, b_ref, a_ref, o_ref):
    h = jnp.dot(x_ref[...], w_ref[...], preferred_element_type=jnp.float32)
    h = h + b_ref[...]
    o_ref[...] = jnp.where(h >= 0.0, h, a_ref[...] * h)


def _enc(x, w, b, a):
    return pl.pallas_call(
        _enc_block,
        grid=(_N // _ROWS,),
        in_specs=[
            pl.BlockSpec((_ROWS, _D), lambda i: (i, 0)),
            pl.BlockSpec((_D, _H), lambda i: (0, 0)),
            pl.BlockSpec((1, _H), lambda i: (0, 0)),
            pl.BlockSpec((1, _H), lambda i: (0, 0)),
        ],
        out_specs=pl.BlockSpec((_ROWS, _H), lambda i: (i, 0)),
        out_shape=jax.ShapeDtypeStruct((_N, _H), jnp.float32),
    )(x, w, b, a)


def _mlp_block(h_ref, ag_ref, w1_ref, b1_ref, w2_ref, b2_ref, g_ref, be_ref,
               o_ref):
    t = h_ref[...] + ag_ref[0] + ag_ref[1]
    t = jnp.maximum(jnp.dot(t, w1_ref[...], preferred_element_type=jnp.float32)
                    + b1_ref[...], 0.0)
    t = jnp.maximum(jnp.dot(t, w2_ref[...], preferred_element_type=jnp.float32)
                    + b2_ref[...], 0.0)
    o_ref[...] = t * g_ref[...] + be_ref[...]


def _mlp(h, ag, w1, b1, w2, b2, gamma, beta):
    return pl.pallas_call(
        _mlp_block,
        grid=(_N // _ROWS,),
        in_specs=[
            pl.BlockSpec((_ROWS, _H), lambda i: (i, 0)),
            pl.BlockSpec((_NC, _ROWS, _H), lambda i: (0, i, 0)),
            pl.BlockSpec((_H, _H), lambda i: (0, 0)),
            pl.BlockSpec((1, _H), lambda i: (0, 0)),
            pl.BlockSpec((_H, _H), lambda i: (0, 0)),
            pl.BlockSpec((1, _H), lambda i: (0, 0)),
            pl.BlockSpec((1, _H), lambda i: (0, 0)),
            pl.BlockSpec((1, _H), lambda i: (0, 0)),
        ],
        out_specs=pl.BlockSpec((_ROWS, _H), lambda i: (i, 0)),
        out_shape=jax.ShapeDtypeStruct((_N, _H), jnp.float32),
    )(h, ag, w1, b1, w2, b2, gamma, beta)


def _cls_block(g_ref, w1_ref, b1_ref, w2_ref, b2_ref, o_ref):
    g = g_ref[0] + g_ref[1]
    t = jnp.maximum(jnp.dot(g, w1_ref[...], preferred_element_type=jnp.float32)
                    + b1_ref[...], 0.0)
    o_ref[...] = jnp.dot(t, w2_ref[...],
                         preferred_element_type=jnp.float32) + b2_ref[...]


def _cls(gparts, w1, b1, w2p, b2p):
    return pl.pallas_call(
        _cls_block,
        in_specs=[
            pl.BlockSpec((_NC, _B, _H), lambda: (0, 0, 0)),
            pl.BlockSpec((_H, _H), lambda: (0, 0)),
            pl.BlockSpec((1, _H), lambda: (0, 0)),
            pl.BlockSpec((_H, _H), lambda: (0, 0)),
            pl.BlockSpec((1, _H), lambda: (0, 0)),
        ],
        out_specs=pl.BlockSpec((_B, _H), lambda: (0, 0)),
        out_shape=jax.ShapeDtypeStruct((_B, _H), jnp.float32),
    )(gparts, w1, b1, w2p, b2p)


def kernel(x, edge_index, assign_index, batch, enc_W, enc_b, prelu_a,
           conv0_W1, conv0_b1, conv0_W2, conv0_b2, conv0_gamma, conv0_beta,
           conv1_W1, conv1_b1, conv1_W2, conv1_b2, conv1_gamma, conv1_beta,
           cls_W1, cls_b1, cls_W2, cls_b2):
    srcp = jnp.pad(edge_index[0].reshape(_NW, _EPT),
                   ((0, 0), (0, _EPAD - _EPT)))
    dstp = jnp.pad(edge_index[1].reshape(_NW, _EPT),
                   ((0, 0), (0, _EPAD - _EPT)), constant_values=_N)
    idxcat = jnp.stack([srcp.reshape(_NW, _NCHUNK, _CH),
                        dstp.reshape(_NW, _NCHUNK, _CH)],
                       axis=2).reshape(_NW, 2 * _NCHUNK, _CH)
    zeros = jnp.zeros((_ZR, _H), jnp.float32)

    h = _enc(x, enc_W, enc_b.reshape(1, _H), prelu_a.reshape(1, _H))
    ag = _edge_aggr(h, idxcat, zeros)
    h = _mlp(h, ag, conv0_W1, conv0_b1.reshape(1, _H),
             conv0_W2, conv0_b2.reshape(1, _H),
             conv0_gamma.reshape(1, _H), conv0_beta.reshape(1, _H))
    ag = _edge_aggr(h, idxcat, zeros)
    h = _mlp(h, ag, conv1_W1, conv1_b1.reshape(1, _H),
             conv1_W2, conv1_b2.reshape(1, _H),
             conv1_gamma.reshape(1, _H), conv1_beta.reshape(1, _H))

    gparts = _pool(h, assign_index[0], assign_index[1], batch, zeros)

    w2p = jnp.pad(cls_W2, ((0, 0), (0, _H - _C)))
    b2p = jnp.pad(cls_b2, (0, _H - _C)).reshape(1, _H)
    out = _cls(gparts, cls_W1, cls_b1.reshape(1, _H), w2p, b2p)
    return out[:, :_C]


# speedup vs baseline: 1.0017x; 1.0017x over previous
import functools

import jax
import jax.numpy as jnp
from jax import lax
from jax.experimental import pallas as pl
from jax.experimental.pallas import tpu as pltpu
from jax.experimental.pallas import tpu_sc as plsc

_N, _E, _D, _H, _B, _C = 10000, 320000, 128, 128, 8, 2
_NC, _NS = 2, 16
_NW = _NC * _NS
_CH = 128
_EPT = _E // _NW
_NCHUNK = 79
_EPAD = _NCHUNK * _CH
_NACC = _N + 8
_RPT = 624
_ZR = _CH
_RTAIL = _N - _NS * _RPT

_PPT = 312
_PCH = 104
_PTAIL = _N - _NW * _PPT

_sc_mesh = plsc.VectorSubcoreMesh(core_axis_name="c", subcore_axis_name="s")


def _edge_aggr_body(h_hbm, src_hbm, dst_hbm, zeros_hbm, out_hbm,
                    acc_sh, src_v, dst_v, rows, gsems, ssems):
    c = lax.axis_index("c")
    s = lax.axis_index("s")
    wid = c * _NS + s

    pltpu.sync_copy(zeros_hbm, rows[0])
    for k in range(4):
        pltpu.sync_copy(rows[0], acc_sh.at[pl.ds(s * _RPT + k * _CH, _CH)])
    pltpu.sync_copy(rows[0].at[pl.ds(0, 112)],
                    acc_sh.at[pl.ds(s * _RPT + 4 * _CH, 112)])

    @pl.when(s == 0)
    def _zero_tail():
        pltpu.sync_copy(rows[0].at[pl.ds(0, _RTAIL)],
                        acc_sh.at[pl.ds(_NS * _RPT, _RTAIL)])

    plsc.subcore_barrier()

    def gather(j, b):
        pltpu.async_copy(h_hbm.at[src_v.at[j]], rows[b], gsems[b])

    def gather_wait(j, b):
        pltpu.make_async_copy(h_hbm.at[src_v.at[j]], rows[b],
                              gsems[b]).wait()

    def scatter(j, b):
        pltpu.async_copy(rows[b], acc_sh.at[dst_v.at[j]], ssems[b],
                         add=True)

    def scatter_wait(j, b):
        pltpu.make_async_copy(rows[b], acc_sh.at[dst_v.at[j]],
                              ssems[b]).wait()

    def run_phase(nchunks):
        npairs = (nchunks - 2) // 2

        def body(k, carry):
            a = 2 * k
            gather_wait(a, 0)
            scatter(a, 0)
            gather_wait(a + 1, 1)
            scatter(a + 1, 1)
            scatter_wait(a, 0)
            gather(a + 2, 0)
            scatter_wait(a + 1, 1)
            gather(a + 3, 1)
            return carry

        gather(0, 0)
        gather(1, 1)
        lax.fori_loop(0, npairs, body, 0)
        a = 2 * npairs
        gather_wait(a, 0)
        scatter(a, 0)
        gather_wait(a + 1, 1)
        scatter(a + 1, 1)
        scatter_wait(a, 0)
        if nchunks % 2:
            gather(a + 2, 0)
            gather_wait(a + 2, 0)
            scatter(a + 2, 0)
            scatter_wait(a + 2, 0)
        scatter_wait(a + 1, 1)

    pltpu.sync_copy(src_hbm.at[wid, pl.ds(0, 40)], src_v)
    pltpu.sync_copy(dst_hbm.at[wid, pl.ds(0, 40)], dst_v)
    run_phase(40)
    pltpu.sync_copy(src_hbm.at[wid, pl.ds(40, 39)], src_v.at[pl.ds(0, 39)])
    pltpu.sync_copy(dst_hbm.at[wid, pl.ds(40, 39)], dst_v.at[pl.ds(0, 39)])
    run_phase(39)

    plsc.subcore_barrier()

    for k in range(4):
        r0 = s * _RPT + k * _CH
        pltpu.sync_copy(acc_sh.at[pl.ds(r0, _CH)], rows[0])
        pltpu.sync_copy(rows[0], out_hbm.at[c, pl.ds(r0, _CH)])
    r1 = s * _RPT + 4 * _CH
    pltpu.sync_copy(acc_sh.at[pl.ds(r1, 112)], rows[0].at[pl.ds(0, 112)])
    pltpu.sync_copy(rows[0].at[pl.ds(0, 112)], out_hbm.at[c, pl.ds(r1, 112)])

    @pl.when(s == 0)
    def _out_tail():
        r0 = _NS * _RPT
        pltpu.sync_copy(acc_sh.at[pl.ds(r0, _RTAIL)],
                        rows[0].at[pl.ds(0, _RTAIL)])
        pltpu.sync_copy(rows[0].at[pl.ds(0, _RTAIL)],
                        out_hbm.at[c, pl.ds(r0, _RTAIL)])


@functools.partial(
    pl.kernel,
    out_type=jax.ShapeDtypeStruct((_NC, _N, _H), jnp.float32),
    mesh=_sc_mesh,
    scratch_types=[
        pltpu.VMEM_SHARED((_NACC, _H), jnp.float32),
        pltpu.VMEM((40, _CH), jnp.int32),
        pltpu.VMEM((40, _CH), jnp.int32),
        pltpu.VMEM((_CH, _H), jnp.float32),
        pltpu.VMEM((_CH, _H), jnp.float32),
        pltpu.SemaphoreType.DMA,
        pltpu.SemaphoreType.DMA,
        pltpu.SemaphoreType.DMA,
        pltpu.SemaphoreType.DMA,
    ],
)
def _edge_aggr(h_hbm, src_hbm, dst_hbm, zeros_hbm, out_hbm,
               acc_sh, src_v, dst_v, rows0, rows1, g0, g1, s0, s1):
    _edge_aggr_body(h_hbm, src_hbm, dst_hbm, zeros_hbm, out_hbm,
                    acc_sh, src_v, dst_v, (rows0, rows1), (g0, g1), (s0, s1))


def _pool_body(h_hbm, a0_hbm, a1_hbm, batch_hbm, zeros_hbm, out_hbm,
               g_sh, a0_v, a1_v, idxb_v, rows_v, zg_v,
               a0t_v, a1t_v, idxbt_v, rowst_v, sem, sem2):
    c = lax.axis_index("c")
    s = lax.axis_index("s")
    wid = c * _NS + s

    @pl.when(s == 0)
    def _init():
        pltpu.sync_copy(zeros_hbm.at[pl.ds(0, _B)], zg_v)
        pltpu.sync_copy(zg_v, g_sh)

    plsc.subcore_barrier()

    base = wid * _PPT
    for j in range(_PPT // _PCH):
        off = base + j * _PCH
        pltpu.sync_copy(a0_hbm.at[pl.ds(off, _PCH)], a0_v)
        pltpu.sync_copy(a1_hbm.at[pl.ds(off, _PCH)], a1_v)
        cp_rows = pltpu.async_copy(h_hbm.at[a1_v], rows_v, sem)
        cp_idx = pltpu.async_copy(batch_hbm.at[a0_v], idxb_v, sem2)
        cp_rows.wait()
        cp_idx.wait()
        pltpu.sync_copy(rows_v, g_sh.at[idxb_v], add=True)

    @pl.when(wid == 0)
    def _tail():
        off = _NW * _PPT
        pltpu.sync_copy(a0_hbm.at[pl.ds(off, _PTAIL)], a0t_v)
        pltpu.sync_copy(a1_hbm.at[pl.ds(off, _PTAIL)], a1t_v)
        cp_rows = pltpu.async_copy(h_hbm.at[a1t_v], rowst_v, sem)
        cp_idx = pltpu.async_copy(batch_hbm.at[a0t_v], idxbt_v, sem2)
        cp_rows.wait()
        cp_idx.wait()
        pltpu.sync_copy(rowst_v, g_sh.at[idxbt_v], add=True)

    plsc.subcore_barrier()

    @pl.when(s == 0)
    def _out():
        pltpu.sync_copy(g_sh, zg_v)
        pltpu.sync_copy(zg_v, out_hbm.at[c])


@functools.partial(
    pl.kernel,
    out_type=jax.ShapeDtypeStruct((_NC, _B, _H), jnp.float32),
    mesh=_sc_mesh,
    scratch_types=[
        pltpu.VMEM_SHARED((_B, _H), jnp.float32),
        pltpu.VMEM((_PCH,), jnp.int32),
        pltpu.VMEM((_PCH,), jnp.int32),
        pltpu.VMEM((_PCH,), jnp.int32),
        pltpu.VMEM((_PCH, _H), jnp.float32),
        pltpu.VMEM((_B, _H), jnp.float32),
        pltpu.VMEM((_PTAIL,), jnp.int32),
        pltpu.VMEM((_PTAIL,), jnp.int32),
        pltpu.VMEM((_PTAIL,), jnp.int32),
        pltpu.VMEM((_PTAIL, _H), jnp.float32),
        pltpu.SemaphoreType.DMA,
        pltpu.SemaphoreType.DMA,
    ],
)
def _pool(h_hbm, a0_hbm, a1_hbm, batch_hbm, zeros_hbm, out_hbm,
          g_sh, a0_v, a1_v, idxb_v, rows_v, zg_v,
          a0t_v, a1t_v, idxbt_v, rowst_v, sem, sem2):
    _pool_body(h_hbm, a0_hbm, a1_hbm, batch_hbm, zeros_hbm, out_hbm,
               g_sh, a0_v, a1_v, idxb_v, rows_v, zg_v,
               a0t_v, a1t_v, idxbt_v, rowst_v, sem, sem2)


_ROWS = 1000


def _enc_block(x_ref, w_ref, b_ref, a_ref, o_ref):
    h = jnp.dot(x_ref[...], w_ref[...], preferred_element_type=jnp.float32)
    h = h + b_ref[...]
    o_ref[...] = jnp.where(h >= 0.0, h, a_ref[...] * h)


def _enc(x, w, b, a):
    return pl.pallas_call(
        _enc_block,
        grid=(_N // _ROWS,),
        in_specs=[
            pl.BlockSpec((_ROWS, _D), lambda i: (i, 0)),
            pl.BlockSpec((_D, _H), lambda i: (0, 0)),
            pl.BlockSpec((1, _H), lambda i: (0, 0)),
            pl.BlockSpec((1, _H), lambda i: (0, 0)),
        ],
        out_specs=pl.BlockSpec((_ROWS, _H), lambda i: (i, 0)),
        out_shape=jax.ShapeDtypeStruct((_N, _H), jnp.float32),
    )(x, w, b, a)


def _mlp_block(h_ref, ag_ref, w1_ref, b1_ref, w2_ref, b2_ref, g_ref, be_ref,
               o_ref):
    t = h_ref[...] + ag_ref[0] + ag_ref[1]
    t = jnp.maximum(jnp.dot(t, w1_ref[...], preferred_element_type=jnp.float32)
                    + b1_ref[...], 0.0)
    t = jnp.maximum(jnp.dot(t, w2_ref[...], preferred_element_type=jnp.float32)
                    + b2_ref[...], 0.0)
    o_ref[...] = t * g_ref[...] + be_ref[...]


def _mlp(h, ag, w1, b1, w2, b2, gamma, beta):
    return pl.pallas_call(
        _mlp_block,
        grid=(_N // _ROWS,),
        in_specs=[
            pl.BlockSpec((_ROWS, _H), lambda i: (i, 0)),
            pl.BlockSpec((_NC, _ROWS, _H), lambda i: (0, i, 0)),
            pl.BlockSpec((_H, _H), lambda i: (0, 0)),
            pl.BlockSpec((1, _H), lambda i: (0, 0)),
            pl.BlockSpec((_H, _H), lambda i: (0, 0)),
            pl.BlockSpec((1, _H), lambda i: (0, 0)),
            pl.BlockSpec((1, _H), lambda i: (0, 0)),
            pl.BlockSpec((1, _H), lambda i: (0, 0)),
        ],
        out_specs=pl.BlockSpec((_ROWS, _H), lambda i: (i, 0)),
        out_shape=jax.ShapeDtypeStruct((_N, _H), jnp.float32),
    )(h, ag, w1, b1, w2, b2, gamma, beta)


def _cls_block(g_ref, w1_ref, b1_ref, w2_ref, b2_ref, o_ref):
    g = g_ref[0] + g_ref[1]
    t = jnp.maximum(jnp.dot(g, w1_ref[...], preferred_element_type=jnp.float32)
                    + b1_ref[...], 0.0)
    o_ref[...] = jnp.dot(t, w2_ref[...],
                         preferred_element_type=jnp.float32) + b2_ref[...]


def _cls(gparts, w1, b1, w2p, b2p):
    return pl.pallas_call(
        _cls_block,
        in_specs=[
            pl.BlockSpec((_NC, _B, _H), lambda: (0, 0, 0)),
            pl.BlockSpec((_H, _H), lambda: (0, 0)),
            pl.BlockSpec((1, _H), lambda: (0, 0)),
            pl.BlockSpec((_H, _H), lambda: (0, 0)),
            pl.BlockSpec((1, _H), lambda: (0, 0)),
        ],
        out_specs=pl.BlockSpec((_B, _H), lambda: (0, 0)),
        out_shape=jax.ShapeDtypeStruct((_B, _H), jnp.float32),
    )(gparts, w1, b1, w2p, b2p)


def kernel(x, edge_index, assign_index, batch, enc_W, enc_b, prelu_a,
           conv0_W1, conv0_b1, conv0_W2, conv0_b2, conv0_gamma, conv0_beta,
           conv1_W1, conv1_b1, conv1_W2, conv1_b2, conv1_gamma, conv1_beta,
           cls_W1, cls_b1, cls_W2, cls_b2):
    srcp = jnp.pad(edge_index[0].reshape(_NW, _EPT),
                   ((0, 0), (0, _EPAD - _EPT)))
    dstp = jnp.pad(edge_index[1].reshape(_NW, _EPT),
                   ((0, 0), (0, _EPAD - _EPT)), constant_values=_N)
    src3 = srcp.reshape(_NW, _NCHUNK, _CH)
    dst3 = dstp.reshape(_NW, _NCHUNK, _CH)
    zeros = jnp.zeros((_ZR, _H), jnp.float32)

    h = _enc(x, enc_W, enc_b.reshape(1, _H), prelu_a.reshape(1, _H))
    ag = _edge_aggr(h, src3, dst3, zeros)
    h = _mlp(h, ag, conv0_W1, conv0_b1.reshape(1, _H),
             conv0_W2, conv0_b2.reshape(1, _H),
             conv0_gamma.reshape(1, _H), conv0_beta.reshape(1, _H))
    ag = _edge_aggr(h, src3, dst3, zeros)
    h = _mlp(h, ag, conv1_W1, conv1_b1.reshape(1, _H),
             conv1_W2, conv1_b2.reshape(1, _H),
             conv1_gamma.reshape(1, _H), conv1_beta.reshape(1, _H))

    gparts = _pool(h, assign_index[0], assign_index[1], batch, zeros)

    w2p = jnp.pad(cls_W2, ((0, 0), (0, _H - _C)))
    b2p = jnp.pad(cls_b2, (0, _H - _C)).reshape(1, _H)
    out = _cls(gparts, cls_W1, cls_b1.reshape(1, _H), w2p, b2p)
    return out[:, :_C]


# speedup vs baseline: 1.0027x; 1.0009x over previous
import functools

import jax
import jax.numpy as jnp
from jax import lax
from jax.experimental import pallas as pl
from jax.experimental.pallas import tpu as pltpu
from jax.experimental.pallas import tpu_sc as plsc

_N, _E, _D, _H, _B, _C = 10000, 320000, 128, 128, 8, 2
_NC, _NS = 2, 16
_NW = _NC * _NS
_CH = 128
_EPT = _E // _NW
_NCHUNK = 79
_EPAD = _NCHUNK * _CH
_NACC = _N + 112
_RPT = 624
_ZR = _CH
_RTAIL = _N - _NS * _RPT

_PPT = 312
_PCH = 104
_PTAIL = _N - _NW * _PPT

_sc_mesh = plsc.VectorSubcoreMesh(core_axis_name="c", subcore_axis_name="s")


def _edge_aggr_body(h_hbm, src_hbm, dst_hbm, zeros_hbm, out_hbm,
                    acc_sh, src_v, dst_v, rows, gsems, ssems):
    c = lax.axis_index("c")
    s = lax.axis_index("s")
    wid = c * _NS + s

    pltpu.sync_copy(zeros_hbm, rows[0])
    for k in range(4):
        pltpu.sync_copy(rows[0], acc_sh.at[pl.ds(s * _RPT + k * _CH, _CH)])
    pltpu.sync_copy(rows[0].at[pl.ds(0, 112)],
                    acc_sh.at[pl.ds(s * _RPT + 4 * _CH, 112)])

    @pl.when(s == 0)
    def _zero_tail():
        pltpu.sync_copy(rows[0].at[pl.ds(0, _RTAIL)],
                        acc_sh.at[pl.ds(_NS * _RPT, _RTAIL)])

    plsc.subcore_barrier()

    def gather(j, b):
        pltpu.async_copy(h_hbm.at[src_v.at[j]], rows[b], gsems[b])

    def gather_wait(j, b):
        pltpu.make_async_copy(h_hbm.at[src_v.at[j]], rows[b],
                              gsems[b]).wait()

    def scatter(j, b):
        pltpu.async_copy(rows[b], acc_sh.at[dst_v.at[j]], ssems[b],
                         add=True)

    def scatter_wait(j, b):
        pltpu.make_async_copy(rows[b], acc_sh.at[dst_v.at[j]],
                              ssems[b]).wait()

    def run_phase(nchunks):
        npairs = (nchunks - 2) // 2

        def body(k, carry):
            a = 2 * k
            gather_wait(a, 0)
            scatter(a, 0)
            gather_wait(a + 1, 1)
            scatter(a + 1, 1)
            scatter_wait(a, 0)
            gather(a + 2, 0)
            scatter_wait(a + 1, 1)
            gather(a + 3, 1)
            return carry

        gather(0, 0)
        gather(1, 1)
        lax.fori_loop(0, npairs, body, 0)
        a = 2 * npairs
        gather_wait(a, 0)
        scatter(a, 0)
        gather_wait(a + 1, 1)
        scatter(a + 1, 1)
        scatter_wait(a, 0)
        if nchunks % 2:
            gather(a + 2, 0)
            gather_wait(a + 2, 0)
            scatter(a + 2, 0)
            scatter_wait(a + 2, 0)
        scatter_wait(a + 1, 1)

    pltpu.sync_copy(src_hbm.at[wid, pl.ds(0, 40)], src_v)
    pltpu.sync_copy(dst_hbm.at[wid, pl.ds(0, 40)], dst_v)
    run_phase(40)
    pltpu.sync_copy(src_hbm.at[wid, pl.ds(40, 39)], src_v.at[pl.ds(0, 39)])
    pltpu.sync_copy(dst_hbm.at[wid, pl.ds(40, 39)], dst_v.at[pl.ds(0, 39)])
    run_phase(39)

    plsc.subcore_barrier()

    for k in range(4):
        r0 = s * _RPT + k * _CH
        pltpu.sync_copy(acc_sh.at[pl.ds(r0, _CH)], rows[0])
        pltpu.sync_copy(rows[0], out_hbm.at[c, pl.ds(r0, _CH)])
    r1 = s * _RPT + 4 * _CH
    pltpu.sync_copy(acc_sh.at[pl.ds(r1, 112)], rows[0].at[pl.ds(0, 112)])
    pltpu.sync_copy(rows[0].at[pl.ds(0, 112)], out_hbm.at[c, pl.ds(r1, 112)])

    @pl.when(s == 0)
    def _out_tail():
        r0 = _NS * _RPT
        pltpu.sync_copy(acc_sh.at[pl.ds(r0, _RTAIL)],
                        rows[0].at[pl.ds(0, _RTAIL)])
        pltpu.sync_copy(rows[0].at[pl.ds(0, _RTAIL)],
                        out_hbm.at[c, pl.ds(r0, _RTAIL)])


@functools.partial(
    pl.kernel,
    out_type=jax.ShapeDtypeStruct((_NC, _N, _H), jnp.float32),
    mesh=_sc_mesh,
    scratch_types=[
        pltpu.VMEM_SHARED((_NACC, _H), jnp.float32),
        pltpu.VMEM((40, _CH), jnp.int32),
        pltpu.VMEM((40, _CH), jnp.int32),
        pltpu.VMEM((_CH, _H), jnp.float32),
        pltpu.VMEM((_CH, _H), jnp.float32),
        pltpu.SemaphoreType.DMA,
        pltpu.SemaphoreType.DMA,
        pltpu.SemaphoreType.DMA,
        pltpu.SemaphoreType.DMA,
    ],
)
def _edge_aggr(h_hbm, src_hbm, dst_hbm, zeros_hbm, out_hbm,
               acc_sh, src_v, dst_v, rows0, rows1, g0, g1, s0, s1):
    _edge_aggr_body(h_hbm, src_hbm, dst_hbm, zeros_hbm, out_hbm,
                    acc_sh, src_v, dst_v, (rows0, rows1), (g0, g1), (s0, s1))


def _pool_body(h_hbm, a0_hbm, a1_hbm, batch_hbm, zeros_hbm, out_hbm,
               g_sh, a0_v, a1_v, idxb_v, rows_v, zg_v,
               a0t_v, a1t_v, idxbt_v, rowst_v, sem, sem2):
    c = lax.axis_index("c")
    s = lax.axis_index("s")
    wid = c * _NS + s

    @pl.when(s == 0)
    def _init():
        pltpu.sync_copy(zeros_hbm.at[pl.ds(0, _B)], zg_v)
        pltpu.sync_copy(zg_v, g_sh)

    plsc.subcore_barrier()

    base = wid * _PPT
    for j in range(_PPT // _PCH):
        off = base + j * _PCH
        pltpu.sync_copy(a0_hbm.at[pl.ds(off, _PCH)], a0_v)
        pltpu.sync_copy(a1_hbm.at[pl.ds(off, _PCH)], a1_v)
        cp_rows = pltpu.async_copy(h_hbm.at[a1_v], rows_v, sem)
        cp_idx = pltpu.async_copy(batch_hbm.at[a0_v], idxb_v, sem2)
        cp_rows.wait()
        cp_idx.wait()
        pltpu.sync_copy(rows_v, g_sh.at[idxb_v], add=True)

    @pl.when(wid == 0)
    def _tail():
        off = _NW * _PPT
        pltpu.sync_copy(a0_hbm.at[pl.ds(off, _PTAIL)], a0t_v)
        pltpu.sync_copy(a1_hbm.at[pl.ds(off, _PTAIL)], a1t_v)
        cp_rows = pltpu.async_copy(h_hbm.at[a1t_v], rowst_v, sem)
        cp_idx = pltpu.async_copy(batch_hbm.at[a0t_v], idxbt_v, sem2)
        cp_rows.wait()
        cp_idx.wait()
        pltpu.sync_copy(rowst_v, g_sh.at[idxbt_v], add=True)

    plsc.subcore_barrier()

    @pl.when(s == 0)
    def _out():
        pltpu.sync_copy(g_sh, zg_v)
        pltpu.sync_copy(zg_v, out_hbm.at[c])


@functools.partial(
    pl.kernel,
    out_type=jax.ShapeDtypeStruct((_NC, _B, _H), jnp.float32),
    mesh=_sc_mesh,
    scratch_types=[
        pltpu.VMEM_SHARED((_B, _H), jnp.float32),
        pltpu.VMEM((_PCH,), jnp.int32),
        pltpu.VMEM((_PCH,), jnp.int32),
        pltpu.VMEM((_PCH,), jnp.int32),
        pltpu.VMEM((_PCH, _H), jnp.float32),
        pltpu.VMEM((_B, _H), jnp.float32),
        pltpu.VMEM((_PTAIL,), jnp.int32),
        pltpu.VMEM((_PTAIL,), jnp.int32),
        pltpu.VMEM((_PTAIL,), jnp.int32),
        pltpu.VMEM((_PTAIL, _H), jnp.float32),
        pltpu.SemaphoreType.DMA,
        pltpu.SemaphoreType.DMA,
    ],
)
def _pool(h_hbm, a0_hbm, a1_hbm, batch_hbm, zeros_hbm, out_hbm,
          g_sh, a0_v, a1_v, idxb_v, rows_v, zg_v,
          a0t_v, a1t_v, idxbt_v, rowst_v, sem, sem2):
    _pool_body(h_hbm, a0_hbm, a1_hbm, batch_hbm, zeros_hbm, out_hbm,
               g_sh, a0_v, a1_v, idxb_v, rows_v, zg_v,
               a0t_v, a1t_v, idxbt_v, rowst_v, sem, sem2)


_ROWS = 1000


def _enc_block(x_ref, w_ref, b_ref, a_ref, o_ref):
    h = jnp.dot(x_ref[...], w_ref[...], preferred_element_type=jnp.float32)
    h = h + b_ref[...]
    o_ref[...] = jnp.where(h >= 0.0, h, a_ref[...] * h)


def _enc(x, w, b, a):
    return pl.pallas_call(
        _enc_block,
        grid=(_N // _ROWS,),
        in_specs=[
            pl.BlockSpec((_ROWS, _D), lambda i: (i, 0)),
            pl.BlockSpec((_D, _H), lambda i: (0, 0)),
            pl.BlockSpec((1, _H), lambda i: (0, 0)),
            pl.BlockSpec((1, _H), lambda i: (0, 0)),
        ],
        out_specs=pl.BlockSpec((_ROWS, _H), lambda i: (i, 0)),
        out_shape=jax.ShapeDtypeStruct((_N, _H), jnp.float32),
    )(x, w, b, a)


def _mlp_block(h_ref, ag_ref, w1_ref, b1_ref, w2_ref, b2_ref, g_ref, be_ref,
               o_ref):
    t = h_ref[...] + ag_ref[0] + ag_ref[1]
    t = jnp.maximum(jnp.dot(t, w1_ref[...], preferred_element_type=jnp.float32)
                    + b1_ref[...], 0.0)
    t = jnp.maximum(jnp.dot(t, w2_ref[...], preferred_element_type=jnp.float32)
                    + b2_ref[...], 0.0)
    o_ref[...] = t * g_ref[...] + be_ref[...]


def _mlp(h, ag, w1, b1, w2, b2, gamma, beta):
    return pl.pallas_call(
        _mlp_block,
        grid=(_N // _ROWS,),
        in_specs=[
            pl.BlockSpec((_ROWS, _H), lambda i: (i, 0)),
            pl.BlockSpec((_NC, _ROWS, _H), lambda i: (0, i, 0)),
            pl.BlockSpec((_H, _H), lambda i: (0, 0)),
            pl.BlockSpec((1, _H), lambda i: (0, 0)),
            pl.BlockSpec((_H, _H), lambda i: (0, 0)),
            pl.BlockSpec((1, _H), lambda i: (0, 0)),
            pl.BlockSpec((1, _H), lambda i: (0, 0)),
            pl.BlockSpec((1, _H), lambda i: (0, 0)),
        ],
        out_specs=pl.BlockSpec((_ROWS, _H), lambda i: (i, 0)),
        out_shape=jax.ShapeDtypeStruct((_N, _H), jnp.float32),
    )(h, ag, w1, b1, w2, b2, gamma, beta)


def _cls_block(g_ref, w1_ref, b1_ref, w2_ref, b2_ref, o_ref):
    g = g_ref[0] + g_ref[1]
    t = jnp.maximum(jnp.dot(g, w1_ref[...], preferred_element_type=jnp.float32)
                    + b1_ref[...], 0.0)
    o_ref[...] = jnp.dot(t, w2_ref[...],
                         preferred_element_type=jnp.float32) + b2_ref[...]


def _cls(gparts, w1, b1, w2p, b2p):
    return pl.pallas_call(
        _cls_block,
        in_specs=[
            pl.BlockSpec((_NC, _B, _H), lambda: (0, 0, 0)),
            pl.BlockSpec((_H, _H), lambda: (0, 0)),
            pl.BlockSpec((1, _H), lambda: (0, 0)),
            pl.BlockSpec((_H, _H), lambda: (0, 0)),
            pl.BlockSpec((1, _H), lambda: (0, 0)),
        ],
        out_specs=pl.BlockSpec((_B, _H), lambda: (0, 0)),
        out_shape=jax.ShapeDtypeStruct((_B, _H), jnp.float32),
    )(gparts, w1, b1, w2p, b2p)


def kernel(x, edge_index, assign_index, batch, enc_W, enc_b, prelu_a,
           conv0_W1, conv0_b1, conv0_W2, conv0_b2, conv0_gamma, conv0_beta,
           conv1_W1, conv1_b1, conv1_W2, conv1_b2, conv1_gamma, conv1_beta,
           cls_W1, cls_b1, cls_W2, cls_b2):
    srcp = jnp.pad(edge_index[0].reshape(_NW, _EPT),
                   ((0, 0), (0, _EPAD - _EPT)))
    padrows = jnp.broadcast_to(_N + jnp.arange(_EPAD - _EPT, dtype=jnp.int32),
                               (_NW, _EPAD - _EPT))
    dstp = jnp.concatenate([edge_index[1].reshape(_NW, _EPT), padrows], axis=1)
    src3 = srcp.reshape(_NW, _NCHUNK, _CH)
    dst3 = dstp.reshape(_NW, _NCHUNK, _CH)
    zeros = jnp.zeros((_ZR, _H), jnp.float32)

    h = _enc(x, enc_W, enc_b.reshape(1, _H), prelu_a.reshape(1, _H))
    ag = _edge_aggr(h, src3, dst3, zeros)
    h = _mlp(h, ag, conv0_W1, conv0_b1.reshape(1, _H),
             conv0_W2, conv0_b2.reshape(1, _H),
             conv0_gamma.reshape(1, _H), conv0_beta.reshape(1, _H))
    ag = _edge_aggr(h, src3, dst3, zeros)
    h = _mlp(h, ag, conv1_W1, conv1_b1.reshape(1, _H),
             conv1_W2, conv1_b2.reshape(1, _H),
             conv1_gamma.reshape(1, _H), conv1_beta.reshape(1, _H))

    gparts = _pool(h, assign_index[0], assign_index[1], batch, zeros)

    w2p = jnp.pad(cls_W2, ((0, 0), (0, _H - _C)))
    b2p = jnp.pad(cls_b2, (0, _H - _C)).reshape(1, _H)
    out = _cls(gparts, cls_W1, cls_b1.reshape(1, _H), w2p, b2p)
    return out[:, :_C]


# speedup vs baseline: 1.0419x; 1.0391x over previous
import functools

import jax
import jax.numpy as jnp
from jax import lax
from jax.experimental import pallas as pl
from jax.experimental.pallas import tpu as pltpu
from jax.experimental.pallas import tpu_sc as plsc

_N, _E, _D, _H, _B, _C = 10000, 320000, 128, 128, 8, 2
_NC, _NS = 2, 16
_NW = _NC * _NS
_CH = 128
_EPT = _E // _NW
_NCHUNK = 79
_EPAD = _NCHUNK * _CH
_NACC = _N + 112
_RPT = 624
_ZR = _CH
_RTAIL = _N - _NS * _RPT

_PPT = 312
_PCH = 104
_PTAIL = _N - _NW * _PPT

_sc_mesh = plsc.VectorSubcoreMesh(core_axis_name="c", subcore_axis_name="s")


def _edge_aggr_body(h_hbm, src_hbm, dst_hbm, zeros_hbm, out_hbm,
                    acc_sh, src_v, dst_v, rows, gsems, ssems):
    c = lax.axis_index("c")
    s = lax.axis_index("s")
    wid = c * _NS + s

    pltpu.sync_copy(zeros_hbm, rows[0])
    for k in range(4):
        pltpu.sync_copy(rows[0], acc_sh.at[pl.ds(s * _RPT + k * _CH, _CH)])
    pltpu.sync_copy(rows[0].at[pl.ds(0, 112)],
                    acc_sh.at[pl.ds(s * _RPT + 4 * _CH, 112)])

    @pl.when(s == 0)
    def _zero_tail():
        pltpu.sync_copy(rows[0].at[pl.ds(0, _RTAIL)],
                        acc_sh.at[pl.ds(_NS * _RPT, _RTAIL)])

    plsc.subcore_barrier()

    def gather(j, b):
        pltpu.async_copy(h_hbm.at[src_v.at[j]], rows[b], gsems[b])

    def gather_wait(j, b):
        pltpu.make_async_copy(h_hbm.at[src_v.at[j]], rows[b],
                              gsems[b]).wait()

    def scatter(j, b):
        pltpu.async_copy(rows[b], acc_sh.at[dst_v.at[j]], ssems[b],
                         add=True)

    def scatter_wait(j, b):
        pltpu.make_async_copy(rows[b], acc_sh.at[dst_v.at[j]],
                              ssems[b]).wait()

    def run_phase(nchunks):
        m = (nchunks - 2) // 2

        def body(k, carry):
            a = 2 * k + 1
            gather_wait(a, 1)
            scatter(a, 1)
            scatter_wait(a - 1, 0)
            gather(a + 1, 0)
            gather_wait(a + 1, 0)
            scatter(a + 1, 0)
            scatter_wait(a, 1)
            gather(a + 2, 1)
            return carry

        gather(0, 0)
        gather(1, 1)
        gather_wait(0, 0)
        scatter(0, 0)
        lax.fori_loop(0, m, body, 0)
        a = 2 * m + 1
        gather_wait(a, 1)
        scatter(a, 1)
        scatter_wait(a - 1, 0)
        if nchunks % 2:
            gather(a + 1, 0)
            gather_wait(a + 1, 0)
            scatter(a + 1, 0)
            scatter_wait(a, 1)
            scatter_wait(a + 1, 0)
        else:
            scatter_wait(a, 1)

    pltpu.sync_copy(src_hbm.at[wid, pl.ds(0, 40)], src_v)
    pltpu.sync_copy(dst_hbm.at[wid, pl.ds(0, 40)], dst_v)
    run_phase(40)
    pltpu.sync_copy(src_hbm.at[wid, pl.ds(40, 39)], src_v.at[pl.ds(0, 39)])
    pltpu.sync_copy(dst_hbm.at[wid, pl.ds(40, 39)], dst_v.at[pl.ds(0, 39)])
    run_phase(39)

    plsc.subcore_barrier()

    for k in range(4):
        r0 = s * _RPT + k * _CH
        pltpu.sync_copy(acc_sh.at[pl.ds(r0, _CH)], rows[0])
        pltpu.sync_copy(rows[0], out_hbm.at[c, pl.ds(r0, _CH)])
    r1 = s * _RPT + 4 * _CH
    pltpu.sync_copy(acc_sh.at[pl.ds(r1, 112)], rows[0].at[pl.ds(0, 112)])
    pltpu.sync_copy(rows[0].at[pl.ds(0, 112)], out_hbm.at[c, pl.ds(r1, 112)])

    @pl.when(s == 0)
    def _out_tail():
        r0 = _NS * _RPT
        pltpu.sync_copy(acc_sh.at[pl.ds(r0, _RTAIL)],
                        rows[0].at[pl.ds(0, _RTAIL)])
        pltpu.sync_copy(rows[0].at[pl.ds(0, _RTAIL)],
                        out_hbm.at[c, pl.ds(r0, _RTAIL)])


@functools.partial(
    pl.kernel,
    out_type=jax.ShapeDtypeStruct((_NC, _N, _H), jnp.float32),
    mesh=_sc_mesh,
    scratch_types=[
        pltpu.VMEM_SHARED((_NACC, _H), jnp.float32),
        pltpu.VMEM((40, _CH), jnp.int32),
        pltpu.VMEM((40, _CH), jnp.int32),
        pltpu.VMEM((_CH, _H), jnp.float32),
        pltpu.VMEM((_CH, _H), jnp.float32),
        pltpu.SemaphoreType.DMA,
        pltpu.SemaphoreType.DMA,
        pltpu.SemaphoreType.DMA,
        pltpu.SemaphoreType.DMA,
    ],
)
def _edge_aggr(h_hbm, src_hbm, dst_hbm, zeros_hbm, out_hbm,
               acc_sh, src_v, dst_v, rows0, rows1, g0, g1, s0, s1):
    _edge_aggr_body(h_hbm, src_hbm, dst_hbm, zeros_hbm, out_hbm,
                    acc_sh, src_v, dst_v, (rows0, rows1), (g0, g1), (s0, s1))


def _pool_body(h_hbm, a0_hbm, a1_hbm, batch_hbm, zeros_hbm, out_hbm,
               g_sh, a0_v, a1_v, idxb_v, rows_v, zg_v,
               a0t_v, a1t_v, idxbt_v, rowst_v, sem, sem2):
    c = lax.axis_index("c")
    s = lax.axis_index("s")
    wid = c * _NS + s

    @pl.when(s == 0)
    def _init():
        pltpu.sync_copy(zeros_hbm.at[pl.ds(0, _B)], zg_v)
        pltpu.sync_copy(zg_v, g_sh)

    plsc.subcore_barrier()

    base = wid * _PPT
    for j in range(_PPT // _PCH):
        off = base + j * _PCH
        pltpu.sync_copy(a0_hbm.at[pl.ds(off, _PCH)], a0_v)
        pltpu.sync_copy(a1_hbm.at[pl.ds(off, _PCH)], a1_v)
        cp_rows = pltpu.async_copy(h_hbm.at[a1_v], rows_v, sem)
        cp_idx = pltpu.async_copy(batch_hbm.at[a0_v], idxb_v, sem2)
        cp_rows.wait()
        cp_idx.wait()
        pltpu.sync_copy(rows_v, g_sh.at[idxb_v], add=True)

    @pl.when(wid == 0)
    def _tail():
        off = _NW * _PPT
        pltpu.sync_copy(a0_hbm.at[pl.ds(off, _PTAIL)], a0t_v)
        pltpu.sync_copy(a1_hbm.at[pl.ds(off, _PTAIL)], a1t_v)
        cp_rows = pltpu.async_copy(h_hbm.at[a1t_v], rowst_v, sem)
        cp_idx = pltpu.async_copy(batch_hbm.at[a0t_v], idxbt_v, sem2)
        cp_rows.wait()
        cp_idx.wait()
        pltpu.sync_copy(rowst_v, g_sh.at[idxbt_v], add=True)

    plsc.subcore_barrier()

    @pl.when(s == 0)
    def _out():
        pltpu.sync_copy(g_sh, zg_v)
        pltpu.sync_copy(zg_v, out_hbm.at[c])


@functools.partial(
    pl.kernel,
    out_type=jax.ShapeDtypeStruct((_NC, _B, _H), jnp.float32),
    mesh=_sc_mesh,
    scratch_types=[
        pltpu.VMEM_SHARED((_B, _H), jnp.float32),
        pltpu.VMEM((_PCH,), jnp.int32),
        pltpu.VMEM((_PCH,), jnp.int32),
        pltpu.VMEM((_PCH,), jnp.int32),
        pltpu.VMEM((_PCH, _H), jnp.float32),
        pltpu.VMEM((_B, _H), jnp.float32),
        pltpu.VMEM((_PTAIL,), jnp.int32),
        pltpu.VMEM((_PTAIL,), jnp.int32),
        pltpu.VMEM((_PTAIL,), jnp.int32),
        pltpu.VMEM((_PTAIL, _H), jnp.float32),
        pltpu.SemaphoreType.DMA,
        pltpu.SemaphoreType.DMA,
    ],
)
def _pool(h_hbm, a0_hbm, a1_hbm, batch_hbm, zeros_hbm, out_hbm,
          g_sh, a0_v, a1_v, idxb_v, rows_v, zg_v,
          a0t_v, a1t_v, idxbt_v, rowst_v, sem, sem2):
    _pool_body(h_hbm, a0_hbm, a1_hbm, batch_hbm, zeros_hbm, out_hbm,
               g_sh, a0_v, a1_v, idxb_v, rows_v, zg_v,
               a0t_v, a1t_v, idxbt_v, rowst_v, sem, sem2)


_ROWS = 1000


def _enc_block(x_ref, w_ref, b_ref, a_ref, o_ref):
    h = jnp.dot(x_ref[...], w_ref[...], preferred_element_type=jnp.float32)
    h = h + b_ref[...]
    o_ref[...] = jnp.where(h >= 0.0, h, a_ref[...] * h)


def _enc(x, w, b, a):
    return pl.pallas_call(
        _enc_block,
        grid=(_N // _ROWS,),
        in_specs=[
            pl.BlockSpec((_ROWS, _D), lambda i: (i, 0)),
            pl.BlockSpec((_D, _H), lambda i: (0, 0)),
            pl.BlockSpec((1, _H), lambda i: (0, 0)),
            pl.BlockSpec((1, _H), lambda i: (0, 0)),
        ],
        out_specs=pl.BlockSpec((_ROWS, _H), lambda i: (i, 0)),
        out_shape=jax.ShapeDtypeStruct((_N, _H), jnp.float32),
    )(x, w, b, a)


def _mlp_block(h_ref, ag_ref, w1_ref, b1_ref, w2_ref, b2_ref, g_ref, be_ref,
               o_ref):
    t = h_ref[...] + ag_ref[0] + ag_ref[1]
    t = jnp.maximum(jnp.dot(t, w1_ref[...], preferred_element_type=jnp.float32)
                    + b1_ref[...], 0.0)
    t = jnp.maximum(jnp.dot(t, w2_ref[...], preferred_element_type=jnp.float32)
                    + b2_ref[...], 0.0)
    o_ref[...] = t * g_ref[...] + be_ref[...]


def _mlp(h, ag, w1, b1, w2, b2, gamma, beta):
    return pl.pallas_call(
        _mlp_block,
        grid=(_N // _ROWS,),
        in_specs=[
            pl.BlockSpec((_ROWS, _H), lambda i: (i, 0)),
            pl.BlockSpec((_NC, _ROWS, _H), lambda i: (0, i, 0)),
            pl.BlockSpec((_H, _H), lambda i: (0, 0)),
            pl.BlockSpec((1, _H), lambda i: (0, 0)),
            pl.BlockSpec((_H, _H), lambda i: (0, 0)),
            pl.BlockSpec((1, _H), lambda i: (0, 0)),
            pl.BlockSpec((1, _H), lambda i: (0, 0)),
            pl.BlockSpec((1, _H), lambda i: (0, 0)),
        ],
        out_specs=pl.BlockSpec((_ROWS, _H), lambda i: (i, 0)),
        out_shape=jax.ShapeDtypeStruct((_N, _H), jnp.float32),
    )(h, ag, w1, b1, w2, b2, gamma, beta)


def _cls_block(g_ref, w1_ref, b1_ref, w2_ref, b2_ref, o_ref):
    g = g_ref[0] + g_ref[1]
    t = jnp.maximum(jnp.dot(g, w1_ref[...], preferred_element_type=jnp.float32)
                    + b1_ref[...], 0.0)
    o_ref[...] = jnp.dot(t, w2_ref[...],
                         preferred_element_type=jnp.float32) + b2_ref[...]


def _cls(gparts, w1, b1, w2p, b2p):
    return pl.pallas_call(
        _cls_block,
        in_specs=[
            pl.BlockSpec((_NC, _B, _H), lambda: (0, 0, 0)),
            pl.BlockSpec((_H, _H), lambda: (0, 0)),
            pl.BlockSpec((1, _H), lambda: (0, 0)),
            pl.BlockSpec((_H, _H), lambda: (0, 0)),
            pl.BlockSpec((1, _H), lambda: (0, 0)),
        ],
        out_specs=pl.BlockSpec((_B, _H), lambda: (0, 0)),
        out_shape=jax.ShapeDtypeStruct((_B, _H), jnp.float32),
    )(gparts, w1, b1, w2p, b2p)


def kernel(x, edge_index, assign_index, batch, enc_W, enc_b, prelu_a,
           conv0_W1, conv0_b1, conv0_W2, conv0_b2, conv0_gamma, conv0_beta,
           conv1_W1, conv1_b1, conv1_W2, conv1_b2, conv1_gamma, conv1_beta,
           cls_W1, cls_b1, cls_W2, cls_b2):
    srcp = jnp.pad(edge_index[0].reshape(_NW, _EPT),
                   ((0, 0), (0, _EPAD - _EPT)))
    padrows = jnp.broadcast_to(_N + jnp.arange(_EPAD - _EPT, dtype=jnp.int32),
                               (_NW, _EPAD - _EPT))
    dstp = jnp.concatenate([edge_index[1].reshape(_NW, _EPT), padrows], axis=1)
    src3 = srcp.reshape(_NW, _NCHUNK, _CH)
    dst3 = dstp.reshape(_NW, _NCHUNK, _CH)
    zeros = jnp.zeros((_ZR, _H), jnp.float32)

    h = _enc(x, enc_W, enc_b.reshape(1, _H), prelu_a.reshape(1, _H))
    ag = _edge_aggr(h, src3, dst3, zeros)
    h = _mlp(h, ag, conv0_W1, conv0_b1.reshape(1, _H),
             conv0_W2, conv0_b2.reshape(1, _H),
             conv0_gamma.reshape(1, _H), conv0_beta.reshape(1, _H))
    ag = _edge_aggr(h, src3, dst3, zeros)
    h = _mlp(h, ag, conv1_W1, conv1_b1.reshape(1, _H),
             conv1_W2, conv1_b2.reshape(1, _H),
             conv1_gamma.reshape(1, _H), conv1_beta.reshape(1, _H))

    gparts = _pool(h, assign_index[0], assign_index[1], batch, zeros)

    w2p = jnp.pad(cls_W2, ((0, 0), (0, _H - _C)))
    b2p = jnp.pad(cls_b2, (0, _H - _C)).reshape(1, _H)
    out = _cls(gparts, cls_W1, cls_b1.reshape(1, _H), w2p, b2p)
    return out[:, :_C]


# speedup vs baseline: 1.5745x; 1.5111x over previous
import functools

import jax
import jax.numpy as jnp
from jax import lax
from jax.experimental import pallas as pl
from jax.experimental.pallas import tpu as pltpu
from jax.experimental.pallas import tpu_sc as plsc

_N, _E, _D, _H, _B, _C = 10000, 320000, 128, 128, 8, 2
_NC, _NS = 2, 16
_NW = _NC * _NS
_CH = 80
_EPT = _E // _NW
_NCHUNK = _EPT // _CH
_NACC = _N
_RPT = 624
_ZR = _CH
_RTAIL = _N - _NS * _RPT

_PPT = 312
_PCH = 104
_PTAIL = _N - _NW * _PPT

_sc_mesh = plsc.VectorSubcoreMesh(core_axis_name="c", subcore_axis_name="s")


def _edge_aggr_body(h_hbm, src_hbm, dst_hbm, zeros_hbm, out_hbm,
                    acc_sh, src_v, dst_v, rows, gsems, ssems):
    c = lax.axis_index("c")
    s = lax.axis_index("s")
    wid = c * _NS + s

    pltpu.sync_copy(zeros_hbm, rows[0])
    for k in range(7):
        pltpu.sync_copy(rows[0], acc_sh.at[pl.ds(s * _RPT + k * _CH, _CH)])
    pltpu.sync_copy(rows[0].at[pl.ds(0, 64)],
                    acc_sh.at[pl.ds(s * _RPT + 7 * _CH, 64)])

    @pl.when(s == 0)
    def _zero_tail():
        pltpu.sync_copy(rows[0].at[pl.ds(0, _RTAIL)],
                        acc_sh.at[pl.ds(_NS * _RPT, _RTAIL)])

    plsc.subcore_barrier()

    def gather(j, b):
        pltpu.async_copy(h_hbm.at[src_v.at[j]], rows[b], gsems[b])

    def gather_wait(j, b):
        pltpu.make_async_copy(h_hbm.at[src_v.at[j]], rows[b],
                              gsems[b]).wait()

    def scatter(j, b):
        pltpu.async_copy(rows[b], acc_sh.at[dst_v.at[j]], ssems[b],
                         add=True)

    def scatter_wait(j, b):
        pltpu.make_async_copy(rows[b], acc_sh.at[dst_v.at[j]],
                              ssems[b]).wait()

    def run_phase(nchunks):
        m = (nchunks - 2) // 2

        def body(k, carry):
            a = 2 * k + 1
            gather_wait(a, 1)
            scatter(a, 1)
            scatter_wait(a - 1, 0)
            gather(a + 1, 0)
            gather_wait(a + 1, 0)
            scatter(a + 1, 0)
            scatter_wait(a, 1)
            gather(a + 2, 1)
            return carry

        gather(0, 0)
        gather(1, 1)
        gather_wait(0, 0)
        scatter(0, 0)
        lax.fori_loop(0, m, body, 0)
        a = 2 * m + 1
        gather_wait(a, 1)
        scatter(a, 1)
        scatter_wait(a - 1, 0)
        if nchunks % 2:
            gather(a + 1, 0)
            gather_wait(a + 1, 0)
            scatter(a + 1, 0)
            scatter_wait(a, 1)
            scatter_wait(a + 1, 0)
        else:
            scatter_wait(a, 1)

    pltpu.sync_copy(src_hbm.at[wid, pl.ds(0, 64)], src_v)
    pltpu.sync_copy(dst_hbm.at[wid, pl.ds(0, 64)], dst_v)
    run_phase(64)
    pltpu.sync_copy(src_hbm.at[wid, pl.ds(64, 61)], src_v.at[pl.ds(0, 61)])
    pltpu.sync_copy(dst_hbm.at[wid, pl.ds(64, 61)], dst_v.at[pl.ds(0, 61)])
    run_phase(61)

    plsc.subcore_barrier()

    for k in range(7):
        r0 = s * _RPT + k * _CH
        pltpu.sync_copy(acc_sh.at[pl.ds(r0, _CH)], rows[0])
        pltpu.sync_copy(rows[0], out_hbm.at[c, pl.ds(r0, _CH)])
    r1 = s * _RPT + 7 * _CH
    pltpu.sync_copy(acc_sh.at[pl.ds(r1, 64)], rows[0].at[pl.ds(0, 64)])
    pltpu.sync_copy(rows[0].at[pl.ds(0, 64)], out_hbm.at[c, pl.ds(r1, 64)])

    @pl.when(s == 0)
    def _out_tail():
        r0 = _NS * _RPT
        pltpu.sync_copy(acc_sh.at[pl.ds(r0, _RTAIL)],
                        rows[0].at[pl.ds(0, _RTAIL)])
        pltpu.sync_copy(rows[0].at[pl.ds(0, _RTAIL)],
                        out_hbm.at[c, pl.ds(r0, _RTAIL)])


@functools.partial(
    pl.kernel,
    out_type=jax.ShapeDtypeStruct((_NC, _N, _H), jnp.float32),
    mesh=_sc_mesh,
    scratch_types=[
        pltpu.VMEM_SHARED((_NACC, _H), jnp.float32),
        pltpu.VMEM((64, _CH), jnp.int32),
        pltpu.VMEM((64, _CH), jnp.int32),
        pltpu.VMEM((_CH, _H), jnp.float32),
        pltpu.VMEM((_CH, _H), jnp.float32),
        pltpu.SemaphoreType.DMA,
        pltpu.SemaphoreType.DMA,
        pltpu.SemaphoreType.DMA,
        pltpu.SemaphoreType.DMA,
    ],
)
def _edge_aggr(h_hbm, src_hbm, dst_hbm, zeros_hbm, out_hbm,
               acc_sh, src_v, dst_v, rows0, rows1, g0, g1, s0, s1):
    _edge_aggr_body(h_hbm, src_hbm, dst_hbm, zeros_hbm, out_hbm,
                    acc_sh, src_v, dst_v, (rows0, rows1), (g0, g1), (s0, s1))


def _pool_body(h_hbm, a0_hbm, a1_hbm, batch_hbm, zeros_hbm, out_hbm,
               g_sh, a0_v, a1_v, idxb_v, rows_v, zg_v,
               a0t_v, a1t_v, idxbt_v, rowst_v, sem, sem2):
    c = lax.axis_index("c")
    s = lax.axis_index("s")
    wid = c * _NS + s

    @pl.when(s == 0)
    def _init():
        pltpu.sync_copy(zeros_hbm.at[pl.ds(0, _B)], zg_v)
        pltpu.sync_copy(zg_v, g_sh)

    plsc.subcore_barrier()

    base = wid * _PPT
    for j in range(_PPT // _PCH):
        off = base + j * _PCH
        pltpu.sync_copy(a0_hbm.at[pl.ds(off, _PCH)], a0_v)
        pltpu.sync_copy(a1_hbm.at[pl.ds(off, _PCH)], a1_v)
        cp_rows = pltpu.async_copy(h_hbm.at[a1_v], rows_v, sem)
        cp_idx = pltpu.async_copy(batch_hbm.at[a0_v], idxb_v, sem2)
        cp_rows.wait()
        cp_idx.wait()
        pltpu.sync_copy(rows_v, g_sh.at[idxb_v], add=True)

    @pl.when(wid == 0)
    def _tail():
        off = _NW * _PPT
        pltpu.sync_copy(a0_hbm.at[pl.ds(off, _PTAIL)], a0t_v)
        pltpu.sync_copy(a1_hbm.at[pl.ds(off, _PTAIL)], a1t_v)
        cp_rows = pltpu.async_copy(h_hbm.at[a1t_v], rowst_v, sem)
        cp_idx = pltpu.async_copy(batch_hbm.at[a0t_v], idxbt_v, sem2)
        cp_rows.wait()
        cp_idx.wait()
        pltpu.sync_copy(rowst_v, g_sh.at[idxbt_v], add=True)

    plsc.subcore_barrier()

    @pl.when(s == 0)
    def _out():
        pltpu.sync_copy(g_sh, zg_v)
        pltpu.sync_copy(zg_v, out_hbm.at[c])


@functools.partial(
    pl.kernel,
    out_type=jax.ShapeDtypeStruct((_NC, _B, _H), jnp.float32),
    mesh=_sc_mesh,
    scratch_types=[
        pltpu.VMEM_SHARED((_B, _H), jnp.float32),
        pltpu.VMEM((_PCH,), jnp.int32),
        pltpu.VMEM((_PCH,), jnp.int32),
        pltpu.VMEM((_PCH,), jnp.int32),
        pltpu.VMEM((_PCH, _H), jnp.float32),
        pltpu.VMEM((_B, _H), jnp.float32),
        pltpu.VMEM((_PTAIL,), jnp.int32),
        pltpu.VMEM((_PTAIL,), jnp.int32),
        pltpu.VMEM((_PTAIL,), jnp.int32),
        pltpu.VMEM((_PTAIL, _H), jnp.float32),
        pltpu.SemaphoreType.DMA,
        pltpu.SemaphoreType.DMA,
    ],
)
def _pool(h_hbm, a0_hbm, a1_hbm, batch_hbm, zeros_hbm, out_hbm,
          g_sh, a0_v, a1_v, idxb_v, rows_v, zg_v,
          a0t_v, a1t_v, idxbt_v, rowst_v, sem, sem2):
    _pool_body(h_hbm, a0_hbm, a1_hbm, batch_hbm, zeros_hbm, out_hbm,
               g_sh, a0_v, a1_v, idxb_v, rows_v, zg_v,
               a0t_v, a1t_v, idxbt_v, rowst_v, sem, sem2)


_ROWS = 1000


def _enc_block(x_ref, w_ref, b_ref, a_ref, o_ref):
    h = jnp.dot(x_ref[...], w_ref[...], preferred_element_type=jnp.float32)
    h = h + b_ref[...]
    o_ref[...] = jnp.where(h >= 0.0, h, a_ref[...] * h)


def _enc(x, w, b, a):
    return pl.pallas_call(
        _enc_block,
        grid=(_N // _ROWS,),
        in_specs=[
            pl.BlockSpec((_ROWS, _D), lambda i: (i, 0)),
            pl.BlockSpec((_D, _H), lambda i: (0, 0)),
            pl.BlockSpec((1, _H), lambda i: (0, 0)),
            pl.BlockSpec((1, _H), lambda i: (0, 0)),
        ],
        out_specs=pl.BlockSpec((_ROWS, _H), lambda i: (i, 0)),
        out_shape=jax.ShapeDtypeStruct((_N, _H), jnp.float32),
    )(x, w, b, a)


def _mlp_block(h_ref, ag_ref, w1_ref, b1_ref, w2_ref, b2_ref, g_ref, be_ref,
               o_ref):
    t = h_ref[...] + ag_ref[0] + ag_ref[1]
    t = jnp.maximum(jnp.dot(t, w1_ref[...], preferred_element_type=jnp.float32)
                    + b1_ref[...], 0.0)
    t = jnp.maximum(jnp.dot(t, w2_ref[...], preferred_element_type=jnp.float32)
                    + b2_ref[...], 0.0)
    o_ref[...] = t * g_ref[...] + be_ref[...]


def _mlp(h, ag, w1, b1, w2, b2, gamma, beta):
    return pl.pallas_call(
        _mlp_block,
        grid=(_N // _ROWS,),
        in_specs=[
            pl.BlockSpec((_ROWS, _H), lambda i: (i, 0)),
            pl.BlockSpec((_NC, _ROWS, _H), lambda i: (0, i, 0)),
            pl.BlockSpec((_H, _H), lambda i: (0, 0)),
            pl.BlockSpec((1, _H), lambda i: (0, 0)),
            pl.BlockSpec((_H, _H), lambda i: (0, 0)),
            pl.BlockSpec((1, _H), lambda i: (0, 0)),
            pl.BlockSpec((1, _H), lambda i: (0, 0)),
            pl.BlockSpec((1, _H), lambda i: (0, 0)),
        ],
        out_specs=pl.BlockSpec((_ROWS, _H), lambda i: (i, 0)),
        out_shape=jax.ShapeDtypeStruct((_N, _H), jnp.float32),
    )(h, ag, w1, b1, w2, b2, gamma, beta)


def _cls_block(g_ref, w1_ref, b1_ref, w2_ref, b2_ref, o_ref):
    g = g_ref[0] + g_ref[1]
    t = jnp.maximum(jnp.dot(g, w1_ref[...], preferred_element_type=jnp.float32)
                    + b1_ref[...], 0.0)
    o_ref[...] = jnp.dot(t, w2_ref[...],
                         preferred_element_type=jnp.float32) + b2_ref[...]


def _cls(gparts, w1, b1, w2p, b2p):
    return pl.pallas_call(
        _cls_block,
        in_specs=[
            pl.BlockSpec((_NC, _B, _H), lambda: (0, 0, 0)),
            pl.BlockSpec((_H, _H), lambda: (0, 0)),
            pl.BlockSpec((1, _H), lambda: (0, 0)),
            pl.BlockSpec((_H, _H), lambda: (0, 0)),
            pl.BlockSpec((1, _H), lambda: (0, 0)),
        ],
        out_specs=pl.BlockSpec((_B, _H), lambda: (0, 0)),
        out_shape=jax.ShapeDtypeStruct((_B, _H), jnp.float32),
    )(gparts, w1, b1, w2p, b2p)


def kernel(x, edge_index, assign_index, batch, enc_W, enc_b, prelu_a,
           conv0_W1, conv0_b1, conv0_W2, conv0_b2, conv0_gamma, conv0_beta,
           conv1_W1, conv1_b1, conv1_W2, conv1_b2, conv1_gamma, conv1_beta,
           cls_W1, cls_b1, cls_W2, cls_b2):
    src3 = edge_index[0].reshape(_NW, _NCHUNK, _CH)
    dst3 = edge_index[1].reshape(_NW, _NCHUNK, _CH)
    zeros = jnp.zeros((_ZR, _H), jnp.float32)

    h = _enc(x, enc_W, enc_b.reshape(1, _H), prelu_a.reshape(1, _H))
    ag = _edge_aggr(h, src3, dst3, zeros)
    h = _mlp(h, ag, conv0_W1, conv0_b1.reshape(1, _H),
             conv0_W2, conv0_b2.reshape(1, _H),
             conv0_gamma.reshape(1, _H), conv0_beta.reshape(1, _H))
    ag = _edge_aggr(h, src3, dst3, zeros)
    h = _mlp(h, ag, conv1_W1, conv1_b1.reshape(1, _H),
             conv1_W2, conv1_b2.reshape(1, _H),
             conv1_gamma.reshape(1, _H), conv1_beta.reshape(1, _H))

    gparts = _pool(h, assign_index[0], assign_index[1], batch, zeros)

    w2p = jnp.pad(cls_W2, ((0, 0), (0, _H - _C)))
    b2p = jnp.pad(cls_b2, (0, _H - _C)).reshape(1, _H)
    out = _cls(gparts, cls_W1, cls_b1.reshape(1, _H), w2p, b2p)
    return out[:, :_C]


# speedup vs baseline: 2.1507x; 1.3660x over previous
import functools

import jax
import jax.numpy as jnp
from jax import lax
from jax.experimental import pallas as pl
from jax.experimental.pallas import tpu as pltpu
from jax.experimental.pallas import tpu_sc as plsc

_N, _E, _D, _H, _B, _C = 10000, 320000, 128, 128, 8, 2
_NC, _NS = 2, 16
_NW = _NC * _NS
_CH = 80
_EPT = _E // _NW
_NCHUNK = _EPT // _CH
_NACC = _N
_RPT = 624
_ZR = _CH
_RTAIL = _N - _NS * _RPT

_PPT = 312
_PCH = 104
_PTAIL = _N - _NW * _PPT

_sc_mesh = plsc.VectorSubcoreMesh(core_axis_name="c", subcore_axis_name="s")


def _edge_aggr_body(h_hbm, src_hbm, dst_hbm, zeros_hbm, out_hbm,
                    acc_sh, src_v, dst_v, rows, gsems, ssems):
    c = lax.axis_index("c")
    s = lax.axis_index("s")
    wid = c * _NS + s

    pltpu.sync_copy(zeros_hbm, rows[0])
    for k in range(7):
        pltpu.sync_copy(rows[0], acc_sh.at[pl.ds(s * _RPT + k * _CH, _CH)])
    pltpu.sync_copy(rows[0].at[pl.ds(0, 64)],
                    acc_sh.at[pl.ds(s * _RPT + 7 * _CH, 64)])

    @pl.when(s == 0)
    def _zero_tail():
        pltpu.sync_copy(rows[0].at[pl.ds(0, _RTAIL)],
                        acc_sh.at[pl.ds(_NS * _RPT, _RTAIL)])

    plsc.subcore_barrier()

    def gather(j, b):
        pltpu.async_copy(h_hbm.at[src_v.at[j]], rows[b], gsems[b])

    def gather_wait(j, b):
        pltpu.make_async_copy(h_hbm.at[src_v.at[j]], rows[b],
                              gsems[b]).wait()

    def scatter(j, b):
        pltpu.async_copy(rows[b], acc_sh.at[dst_v.at[j]], ssems[b],
                         add=True)

    def scatter_wait(j, b):
        pltpu.make_async_copy(rows[b], acc_sh.at[dst_v.at[j]],
                              ssems[b]).wait()

    def run_phase(nchunks):
        m = (nchunks - 4) // 3

        def body(k, carry):
            a = 3 * k + 1
            gather_wait(a, 1)
            scatter(a, 1)
            scatter_wait(a - 1, 0)
            gather(a + 2, 0)
            gather_wait(a + 1, 2)
            scatter(a + 1, 2)
            scatter_wait(a, 1)
            gather(a + 3, 1)
            gather_wait(a + 2, 0)
            scatter(a + 2, 0)
            scatter_wait(a + 1, 2)
            gather(a + 4, 2)
            return carry

        gather(0, 0)
        gather(1, 1)
        gather(2, 2)
        gather_wait(0, 0)
        scatter(0, 0)
        lax.fori_loop(0, m, body, 0)
        for j in range(3 * m + 1, nchunks):
            b = j % 3
            gather_wait(j, b)
            scatter(j, b)
            scatter_wait(j - 1, (j - 1) % 3)
            if j + 2 < nchunks:
                gather(j + 2, (j - 1) % 3)
        scatter_wait(nchunks - 1, (nchunks - 1) % 3)

    pltpu.sync_copy(src_hbm.at[wid, pl.ds(0, 64)], src_v)
    pltpu.sync_copy(dst_hbm.at[wid, pl.ds(0, 64)], dst_v)
    run_phase(64)
    pltpu.sync_copy(src_hbm.at[wid, pl.ds(64, 61)], src_v.at[pl.ds(0, 61)])
    pltpu.sync_copy(dst_hbm.at[wid, pl.ds(64, 61)], dst_v.at[pl.ds(0, 61)])
    run_phase(61)

    plsc.subcore_barrier()

    for k in range(7):
        r0 = s * _RPT + k * _CH
        pltpu.sync_copy(acc_sh.at[pl.ds(r0, _CH)], rows[0])
        pltpu.sync_copy(rows[0], out_hbm.at[c, pl.ds(r0, _CH)])
    r1 = s * _RPT + 7 * _CH
    pltpu.sync_copy(acc_sh.at[pl.ds(r1, 64)], rows[0].at[pl.ds(0, 64)])
    pltpu.sync_copy(rows[0].at[pl.ds(0, 64)], out_hbm.at[c, pl.ds(r1, 64)])

    @pl.when(s == 0)
    def _out_tail():
        r0 = _NS * _RPT
        pltpu.sync_copy(acc_sh.at[pl.ds(r0, _RTAIL)],
                        rows[0].at[pl.ds(0, _RTAIL)])
        pltpu.sync_copy(rows[0].at[pl.ds(0, _RTAIL)],
                        out_hbm.at[c, pl.ds(r0, _RTAIL)])


@functools.partial(
    pl.kernel,
    out_type=jax.ShapeDtypeStruct((_NC, _N, _H), jnp.float32),
    mesh=_sc_mesh,
    scratch_types=[
        pltpu.VMEM_SHARED((_NACC, _H), jnp.float32),
        pltpu.VMEM((64, _CH), jnp.int32),
        pltpu.VMEM((64, _CH), jnp.int32),
        pltpu.VMEM((_CH, _H), jnp.float32),
        pltpu.VMEM((_CH, _H), jnp.float32),
        pltpu.VMEM((_CH, _H), jnp.float32),
        pltpu.SemaphoreType.DMA,
        pltpu.SemaphoreType.DMA,
        pltpu.SemaphoreType.DMA,
        pltpu.SemaphoreType.DMA,
        pltpu.SemaphoreType.DMA,
        pltpu.SemaphoreType.DMA,
    ],
)
def _edge_aggr(h_hbm, src_hbm, dst_hbm, zeros_hbm, out_hbm,
               acc_sh, src_v, dst_v, rows0, rows1, rows2,
               g0, g1, g2, s0, s1, s2):
    _edge_aggr_body(h_hbm, src_hbm, dst_hbm, zeros_hbm, out_hbm,
                    acc_sh, src_v, dst_v, (rows0, rows1, rows2),
                    (g0, g1, g2), (s0, s1, s2))


def _pool_body(h_hbm, a0_hbm, a1_hbm, batch_hbm, zeros_hbm, out_hbm,
               g_sh, a0_v, a1_v, idxb_v, rows_v, zg_v,
               a0t_v, a1t_v, idxbt_v, rowst_v, sem, sem2):
    c = lax.axis_index("c")
    s = lax.axis_index("s")
    wid = c * _NS + s

    @pl.when(s == 0)
    def _init():
        pltpu.sync_copy(zeros_hbm.at[pl.ds(0, _B)], zg_v)
        pltpu.sync_copy(zg_v, g_sh)

    plsc.subcore_barrier()

    base = wid * _PPT
    for j in range(_PPT // _PCH):
        off = base + j * _PCH
        pltpu.sync_copy(a0_hbm.at[pl.ds(off, _PCH)], a0_v)
        pltpu.sync_copy(a1_hbm.at[pl.ds(off, _PCH)], a1_v)
        cp_rows = pltpu.async_copy(h_hbm.at[a1_v], rows_v, sem)
        cp_idx = pltpu.async_copy(batch_hbm.at[a0_v], idxb_v, sem2)
        cp_rows.wait()
        cp_idx.wait()
        pltpu.sync_copy(rows_v, g_sh.at[idxb_v], add=True)

    @pl.when(wid == 0)
    def _tail():
        off = _NW * _PPT
        pltpu.sync_copy(a0_hbm.at[pl.ds(off, _PTAIL)], a0t_v)
        pltpu.sync_copy(a1_hbm.at[pl.ds(off, _PTAIL)], a1t_v)
        cp_rows = pltpu.async_copy(h_hbm.at[a1t_v], rowst_v, sem)
        cp_idx = pltpu.async_copy(batch_hbm.at[a0t_v], idxbt_v, sem2)
        cp_rows.wait()
        cp_idx.wait()
        pltpu.sync_copy(rowst_v, g_sh.at[idxbt_v], add=True)

    plsc.subcore_barrier()

    @pl.when(s == 0)
    def _out():
        pltpu.sync_copy(g_sh, zg_v)
        pltpu.sync_copy(zg_v, out_hbm.at[c])


@functools.partial(
    pl.kernel,
    out_type=jax.ShapeDtypeStruct((_NC, _B, _H), jnp.float32),
    mesh=_sc_mesh,
    scratch_types=[
        pltpu.VMEM_SHARED((_B, _H), jnp.float32),
        pltpu.VMEM((_PCH,), jnp.int32),
        pltpu.VMEM((_PCH,), jnp.int32),
        pltpu.VMEM((_PCH,), jnp.int32),
        pltpu.VMEM((_PCH, _H), jnp.float32),
        pltpu.VMEM((_B, _H), jnp.float32),
        pltpu.VMEM((_PTAIL,), jnp.int32),
        pltpu.VMEM((_PTAIL,), jnp.int32),
        pltpu.VMEM((_PTAIL,), jnp.int32),
        pltpu.VMEM((_PTAIL, _H), jnp.float32),
        pltpu.SemaphoreType.DMA,
        pltpu.SemaphoreType.DMA,
    ],
)
def _pool(h_hbm, a0_hbm, a1_hbm, batch_hbm, zeros_hbm, out_hbm,
          g_sh, a0_v, a1_v, idxb_v, rows_v, zg_v,
          a0t_v, a1t_v, idxbt_v, rowst_v, sem, sem2):
    _pool_body(h_hbm, a0_hbm, a1_hbm, batch_hbm, zeros_hbm, out_hbm,
               g_sh, a0_v, a1_v, idxb_v, rows_v, zg_v,
               a0t_v, a1t_v, idxbt_v, rowst_v, sem, sem2)


_ROWS = 1000


def _enc_block(x_ref, w_ref, b_ref, a_ref, o_ref):
    h = jnp.dot(x_ref[...], w_ref[...], preferred_element_type=jnp.float32)
    h = h + b_ref[...]
    o_ref[...] = jnp.where(h >= 0.0, h, a_ref[...] * h)


def _enc(x, w, b, a):
    return pl.pallas_call(
        _enc_block,
        grid=(_N // _ROWS,),
        in_specs=[
            pl.BlockSpec((_ROWS, _D), lambda i: (i, 0)),
            pl.BlockSpec((_D, _H), lambda i: (0, 0)),
            pl.BlockSpec((1, _H), lambda i: (0, 0)),
            pl.BlockSpec((1, _H), lambda i: (0, 0)),
        ],
        out_specs=pl.BlockSpec((_ROWS, _H), lambda i: (i, 0)),
        out_shape=jax.ShapeDtypeStruct((_N, _H), jnp.float32),
    )(x, w, b, a)


def _mlp_block(h_ref, ag_ref, w1_ref, b1_ref, w2_ref, b2_ref, g_ref, be_ref,
               o_ref):
    t = h_ref[...] + ag_ref[0] + ag_ref[1]
    t = jnp.maximum(jnp.dot(t, w1_ref[...], preferred_element_type=jnp.float32)
                    + b1_ref[...], 0.0)
    t = jnp.maximum(jnp.dot(t, w2_ref[...], preferred_element_type=jnp.float32)
                    + b2_ref[...], 0.0)
    o_ref[...] = t * g_ref[...] + be_ref[...]


def _mlp(h, ag, w1, b1, w2, b2, gamma, beta):
    return pl.pallas_call(
        _mlp_block,
        grid=(_N // _ROWS,),
        in_specs=[
            pl.BlockSpec((_ROWS, _H), lambda i: (i, 0)),
            pl.BlockSpec((_NC, _ROWS, _H), lambda i: (0, i, 0)),
            pl.BlockSpec((_H, _H), lambda i: (0, 0)),
            pl.BlockSpec((1, _H), lambda i: (0, 0)),
            pl.BlockSpec((_H, _H), lambda i: (0, 0)),
            pl.BlockSpec((1, _H), lambda i: (0, 0)),
            pl.BlockSpec((1, _H), lambda i: (0, 0)),
            pl.BlockSpec((1, _H), lambda i: (0, 0)),
        ],
        out_specs=pl.BlockSpec((_ROWS, _H), lambda i: (i, 0)),
        out_shape=jax.ShapeDtypeStruct((_N, _H), jnp.float32),
    )(h, ag, w1, b1, w2, b2, gamma, beta)


def _cls_block(g_ref, w1_ref, b1_ref, w2_ref, b2_ref, o_ref):
    g = g_ref[0] + g_ref[1]
    t = jnp.maximum(jnp.dot(g, w1_ref[...], preferred_element_type=jnp.float32)
                    + b1_ref[...], 0.0)
    o_ref[...] = jnp.dot(t, w2_ref[...],
                         preferred_element_type=jnp.float32) + b2_ref[...]


def _cls(gparts, w1, b1, w2p, b2p):
    return pl.pallas_call(
        _cls_block,
        in_specs=[
            pl.BlockSpec((_NC, _B, _H), lambda: (0, 0, 0)),
            pl.BlockSpec((_H, _H), lambda: (0, 0)),
            pl.BlockSpec((1, _H), lambda: (0, 0)),
            pl.BlockSpec((_H, _H), lambda: (0, 0)),
            pl.BlockSpec((1, _H), lambda: (0, 0)),
        ],
        out_specs=pl.BlockSpec((_B, _H), lambda: (0, 0)),
        out_shape=jax.ShapeDtypeStruct((_B, _H), jnp.float32),
    )(gparts, w1, b1, w2p, b2p)


def kernel(x, edge_index, assign_index, batch, enc_W, enc_b, prelu_a,
           conv0_W1, conv0_b1, conv0_W2, conv0_b2, conv0_gamma, conv0_beta,
           conv1_W1, conv1_b1, conv1_W2, conv1_b2, conv1_gamma, conv1_beta,
           cls_W1, cls_b1, cls_W2, cls_b2):
    src3 = edge_index[0].reshape(_NW, _NCHUNK, _CH)
    dst3 = edge_index[1].reshape(_NW, _NCHUNK, _CH)
    zeros = jnp.zeros((_ZR, _H), jnp.float32)

    h = _enc(x, enc_W, enc_b.reshape(1, _H), prelu_a.reshape(1, _H))
    ag = _edge_aggr(h, src3, dst3, zeros)
    h = _mlp(h, ag, conv0_W1, conv0_b1.reshape(1, _H),
             conv0_W2, conv0_b2.reshape(1, _H),
             conv0_gamma.reshape(1, _H), conv0_beta.reshape(1, _H))
    ag = _edge_aggr(h, src3, dst3, zeros)
    h = _mlp(h, ag, conv1_W1, conv1_b1.reshape(1, _H),
             conv1_W2, conv1_b2.reshape(1, _H),
             conv1_gamma.reshape(1, _H), conv1_beta.reshape(1, _H))

    gparts = _pool(h, assign_index[0], assign_index[1], batch, zeros)

    w2p = jnp.pad(cls_W2, ((0, 0), (0, _H - _C)))
    b2p = jnp.pad(cls_b2, (0, _H - _C)).reshape(1, _H)
    out = _cls(gparts, cls_W1, cls_b1.reshape(1, _H), w2p, b2p)
    return out[:, :_C]
